# Initial kernel scaffold; baseline (speedup 1.0000x reference)
import jax
import jax.numpy as jnp
from jax.experimental import pallas as pl


def kernel(input, W, b, noise):
    raise NotImplementedError("write your pallas kernel here")


# speedup vs baseline: 1.4366x; 1.4366x over previous
import functools

import jax
import jax.numpy as jnp
from jax.experimental import pallas as pl

NUM_EXPERTS = 64
K = 8
HIGH_NEGATIVE = -100000.0
BLK_T = 1024


def _gate_block(x_ref, w_ref, b_ref, noise_ref, o_ref):
    x = x_ref[...]
    w = w_ref[...]
    logits = jax.lax.dot_general(
        x, w, (((1,), (1,)), ((), ())), preferred_element_type=jnp.float32
    )
    logits = logits + b_ref[...] + noise_ref[...]

    n = logits.shape[1]
    iota = jax.lax.broadcasted_iota(jnp.int32, logits.shape, 1)
    v = logits
    v8 = None
    v9 = None
    for i in range(K + 1):
        m = jnp.max(v, axis=1, keepdims=True)
        if i == K - 1:
            v8 = m
        if i == K:
            v9 = m
        if i < K:
            first = jnp.min(jnp.where(v == m, iota, n), axis=1, keepdims=True)
            v = jnp.where(iota == first, -jnp.inf, v)

    quant = v9 + 0.125 * (v8 - v9)
    masked = jnp.where(logits > quant, logits, HIGH_NEGATIVE)
    rm = jnp.max(masked, axis=1, keepdims=True)
    p = jnp.exp(masked - rm)
    o_ref[...] = p / jnp.sum(p, axis=1, keepdims=True)


@functools.partial(jax.jit, static_argnames=())
def kernel(input, W, b, noise):
    tokens, d_model = input.shape
    b2 = b.reshape(1, NUM_EXPERTS)
    grid = (tokens // BLK_T,)
    return pl.pallas_call(
        _gate_block,
        grid=grid,
        in_specs=[
            pl.BlockSpec((BLK_T, d_model), lambda i: (i, 0)),
            pl.BlockSpec((NUM_EXPERTS, d_model), lambda i: (0, 0)),
            pl.BlockSpec((1, NUM_EXPERTS), lambda i: (0, 0)),
            pl.BlockSpec((BLK_T, NUM_EXPERTS), lambda i: (i, 0)),
        ],
        out_specs=pl.BlockSpec((BLK_T, NUM_EXPERTS), lambda i: (i, 0)),
        out_shape=jax.ShapeDtypeStruct((tokens, NUM_EXPERTS), jnp.float32),
    )(input, W, b2, noise)
